# Initial kernel scaffold; baseline (speedup 1.0000x reference)
import jax
import jax.numpy as jnp
from jax.experimental import pallas as pl


def kernel(x, table):
    raise NotImplementedError("write your pallas kernel here")


# speedup vs baseline: 8.6465x; 8.6465x over previous
import functools

import jax
import jax.numpy as jnp
from jax import lax
from jax.experimental import pallas as pl
from jax.experimental.pallas import tpu as pltpu
from jax.experimental.pallas import tpu_sc as plsc

_NC = 2
_NS = 16
_NW = _NC * _NS

_CH = 512


def _emb_lookup(idx, table):
    (N,) = idx.shape
    V, D = table.shape
    b_per_w = N // _NW
    n_chunks = b_per_w // _CH
    mesh = plsc.VectorSubcoreMesh(core_axis_name="c", subcore_axis_name="s")

    @functools.partial(
        pl.kernel,
        mesh=mesh,
        out_type=jax.ShapeDtypeStruct((N, D), jnp.float32),
        scratch_types=[
            pltpu.VMEM((b_per_w,), jnp.int32),
            pltpu.VMEM((_CH, D), jnp.float32),
            pltpu.SemaphoreType.DMA,
        ],
    )
    def body(idx_hbm, table_hbm, out_hbm, idx_v, rows_v, sem):
        wid = lax.axis_index("s") * _NC + lax.axis_index("c")
        base = wid * b_per_w
        pltpu.sync_copy(idx_hbm.at[pl.ds(base, b_per_w)], idx_v)

        def step(c, carry):
            off = pl.multiple_of(c * _CH, _CH)
            pltpu.async_copy(
                table_hbm.at[idx_v.at[pl.ds(off, _CH)]], rows_v, sem
            ).wait()
            pltpu.sync_copy(rows_v, out_hbm.at[pl.ds(base + off, _CH)])
            return carry

        lax.fori_loop(0, n_chunks, step, 0)

    return body(idx, table)


def kernel(x, table):
    B, S = x.shape
    _, D = table.shape
    idx = x.reshape(B * S).astype(jnp.int32)
    out = _emb_lookup(idx, table)
    return out.reshape(B, S, D)


# speedup vs baseline: 9.2063x; 1.0647x over previous
import functools

import jax
import jax.numpy as jnp
from jax import lax
from jax.experimental import pallas as pl
from jax.experimental.pallas import tpu as pltpu
from jax.experimental.pallas import tpu_sc as plsc

_NC = 2
_NS = 16
_NW = _NC * _NS

_CH = 400


def _emb_lookup(idx, table):
    (N,) = idx.shape
    V, D = table.shape
    b_per_w = N // _NW
    n_chunks = b_per_w // _CH
    mesh = plsc.VectorSubcoreMesh(core_axis_name="c", subcore_axis_name="s")

    @functools.partial(
        pl.kernel,
        mesh=mesh,
        out_type=jax.ShapeDtypeStruct((N, D), jnp.float32),
        scratch_types=[
            pltpu.VMEM((b_per_w,), jnp.int32),
            pltpu.VMEM((2, _CH, D), jnp.float32),
            pltpu.SemaphoreType.DMA,
        ],
    )
    def body(idx_hbm, table_hbm, out_hbm, idx_v, rows_v, sem):
        wid = lax.axis_index("s") * _NC + lax.axis_index("c")
        base = wid * b_per_w
        pltpu.sync_copy(idx_hbm.at[pl.ds(base, b_per_w)], idx_v)

        def gather(c, buf):
            off = pl.multiple_of(c * _CH, _CH)
            pltpu.async_copy(
                table_hbm.at[idx_v.at[pl.ds(off, _CH)]], rows_v.at[buf], sem
            )

        gather(0, 0)

        def step(c, carry):
            buf = lax.rem(c, 2)

            @pl.when(c + 1 < n_chunks)
            def _():
                gather(c + 1, 1 - buf)

            pltpu.make_async_copy(
                table_hbm.at[pl.ds(0, _CH)], rows_v.at[buf], sem
            ).wait()
            off = pl.multiple_of(c * _CH, _CH)
            pltpu.sync_copy(rows_v.at[buf], out_hbm.at[pl.ds(base + off, _CH)])
            return carry

        lax.fori_loop(0, n_chunks, step, 0)

    return body(idx, table)


def kernel(x, table):
    B, S = x.shape
    _, D = table.shape
    idx = x.reshape(B * S).astype(jnp.int32)
    out = _emb_lookup(idx, table)
    return out.reshape(B, S, D)


# speedup vs baseline: 9.2506x; 1.0048x over previous
import functools

import jax
import jax.numpy as jnp
from jax import lax
from jax.experimental import pallas as pl
from jax.experimental.pallas import tpu as pltpu
from jax.experimental.pallas import tpu_sc as plsc

_NC = 2
_NS = 16
_NW = _NC * _NS

_CH = 200
_NBUF = 4
_DG = 2
_DS = 2


def _emb_lookup(idx, table):
    (N,) = idx.shape
    V, D = table.shape
    b_per_w = N // _NW
    n_chunks = b_per_w // _CH
    mesh = plsc.VectorSubcoreMesh(core_axis_name="c", subcore_axis_name="s")

    @functools.partial(
        pl.kernel,
        mesh=mesh,
        out_type=jax.ShapeDtypeStruct((N, D), jnp.float32),
        scratch_types=[
            pltpu.VMEM((b_per_w,), jnp.int32),
            pltpu.VMEM((_NBUF, _CH, D), jnp.float32),
            pltpu.SemaphoreType.DMA,
            pltpu.SemaphoreType.DMA,
        ],
    )
    def body(idx_hbm, table_hbm, out_hbm, idx_v, rows_v, sem_g, sem_s):
        wid = lax.axis_index("s") * _NC + lax.axis_index("c")
        base = wid * b_per_w
        pltpu.sync_copy(idx_hbm.at[pl.ds(base, b_per_w)], idx_v)

        def gather(c, buf):
            off = pl.multiple_of(c * _CH, _CH)
            pltpu.async_copy(
                table_hbm.at[idx_v.at[pl.ds(off, _CH)]], rows_v.at[buf], sem_g
            )

        def drain_one(sem):
            pltpu.make_async_copy(
                table_hbm.at[pl.ds(0, _CH)], rows_v.at[0], sem
            ).wait()

        for g in range(_DG):
            gather(g, g % _NBUF)

        def step(c, carry):
            @pl.when(c >= _DS)
            def _():
                drain_one(sem_s)

            @pl.when(c + _DG < n_chunks)
            def _():
                gather(c + _DG, lax.rem(c + _DG, _NBUF))

            drain_one(sem_g)
            buf = lax.rem(c, _NBUF)
            off = pl.multiple_of(c * _CH, _CH)
            pltpu.async_copy(
                rows_v.at[buf], out_hbm.at[pl.ds(base + off, _CH)], sem_s
            )
            return carry

        lax.fori_loop(0, n_chunks, step, 0)
        for _ in range(_DS):
            drain_one(sem_s)

    return body(idx, table)


def kernel(x, table):
    B, S = x.shape
    _, D = table.shape
    idx = x.reshape(B * S).astype(jnp.int32)
    out = _emb_lookup(idx, table)
    return out.reshape(B, S, D)


# speedup vs baseline: 9.2563x; 1.0006x over previous
import functools

import jax
import jax.numpy as jnp
from jax import lax
from jax.experimental import pallas as pl
from jax.experimental.pallas import tpu as pltpu
from jax.experimental.pallas import tpu_sc as plsc

_NC = 2
_NS = 16
_NW = _NC * _NS

_CH = 128
_NBUF = 6
_DG = 3
_DS = 3


def _emb_lookup(idx, table):
    (N,) = idx.shape
    V, D = table.shape
    b_per_w = N // _NW
    n_chunks = b_per_w // _CH
    mesh = plsc.VectorSubcoreMesh(core_axis_name="c", subcore_axis_name="s")

    @functools.partial(
        pl.kernel,
        mesh=mesh,
        out_type=jax.ShapeDtypeStruct((N, D), jnp.float32),
        scratch_types=[
            pltpu.VMEM((b_per_w,), jnp.int32),
            pltpu.VMEM((_NBUF, _CH, D), jnp.float32),
            pltpu.SemaphoreType.DMA,
            pltpu.SemaphoreType.DMA,
        ],
    )
    def body(idx_hbm, table_hbm, out_hbm, idx_v, rows_v, sem_g, sem_s):
        wid = lax.axis_index("s") * _NC + lax.axis_index("c")
        base = wid * b_per_w
        pltpu.sync_copy(idx_hbm.at[pl.ds(base, b_per_w)], idx_v)

        def gather(c, buf):
            off = pl.multiple_of(c * _CH, _CH)
            pltpu.async_copy(
                table_hbm.at[idx_v.at[pl.ds(off, _CH)]], rows_v.at[buf], sem_g
            )

        def drain_one(sem):
            pltpu.make_async_copy(
                table_hbm.at[pl.ds(0, _CH)], rows_v.at[0], sem
            ).wait()

        for g in range(_DG):
            gather(g, g % _NBUF)

        def step(c, carry):
            @pl.when(c >= _DS)
            def _():
                drain_one(sem_s)

            @pl.when(c + _DG < n_chunks)
            def _():
                gather(c + _DG, lax.rem(c + _DG, _NBUF))

            drain_one(sem_g)
            buf = lax.rem(c, _NBUF)
            off = pl.multiple_of(c * _CH, _CH)
            pltpu.async_copy(
                rows_v.at[buf], out_hbm.at[pl.ds(base + off, _CH)], sem_s
            )
            return carry

        lax.fori_loop(0, n_chunks, step, 0)
        for _ in range(_DS):
            drain_one(sem_s)

    return body(idx, table)


def kernel(x, table):
    B, S = x.shape
    _, D = table.shape
    idx = x.reshape(B * S).astype(jnp.int32)
    out = _emb_lookup(idx, table)
    return out.reshape(B, S, D)


# speedup vs baseline: 9.2600x; 1.0004x over previous
import functools

import jax
import jax.numpy as jnp
from jax import lax
from jax.experimental import pallas as pl
from jax.experimental.pallas import tpu as pltpu
from jax.experimental.pallas import tpu_sc as plsc

_NC = 2
_NS = 16
_NW = _NC * _NS

_CH = 200
_NBUF = 4
_DG = 2
_DS = 2


def _emb_lookup(idx, table):
    (N,) = idx.shape
    V, D = table.shape
    b_per_w = N // _NW
    n_chunks = b_per_w // _CH
    mesh = plsc.VectorSubcoreMesh(core_axis_name="c", subcore_axis_name="s")

    @functools.partial(
        pl.kernel,
        mesh=mesh,
        out_type=jax.ShapeDtypeStruct((N, D), jnp.float32),
        scratch_types=[
            pltpu.VMEM((b_per_w,), jnp.int32),
            pltpu.VMEM((_NBUF, _CH, D), jnp.float32),
            pltpu.SemaphoreType.DMA,
            pltpu.SemaphoreType.DMA,
        ],
    )
    def body(idx_hbm, table_hbm, out_hbm, idx_v, rows_v, sem_g, sem_s):
        wid = lax.axis_index("s") * _NC + lax.axis_index("c")
        base = wid * b_per_w
        pltpu.sync_copy(idx_hbm.at[pl.ds(base, b_per_w)], idx_v)

        def gather(c, buf):
            off = pl.multiple_of(c * _CH, _CH)
            pltpu.async_copy(
                table_hbm.at[idx_v.at[pl.ds(off, _CH)]], rows_v.at[buf], sem_g
            )

        def drain_one(sem):
            pltpu.make_async_copy(
                table_hbm.at[pl.ds(0, _CH)], rows_v.at[0], sem
            ).wait()

        for g in range(_DG):
            gather(g, g % _NBUF)

        def step(c, carry):
            @pl.when(c >= _DS)
            def _():
                drain_one(sem_s)

            @pl.when(c + _DG < n_chunks)
            def _():
                gather(c + _DG, lax.rem(c + _DG, _NBUF))

            drain_one(sem_g)
            buf = lax.rem(c, _NBUF)
            off = pl.multiple_of(c * _CH, _CH)
            pltpu.async_copy(
                rows_v.at[buf], out_hbm.at[pl.ds(base + off, _CH)], sem_s
            )
            return carry

        lax.fori_loop(0, n_chunks, step, 0)
        for _ in range(_DS):
            drain_one(sem_s)

    return body(idx, table)


def kernel(x, table):
    B, S = x.shape
    _, D = table.shape
    idx = x.reshape(B * S).astype(jnp.int32)
    out = _emb_lookup(idx, table)
    return out.reshape(B, S, D)
